# Initial kernel scaffold; baseline (speedup 1.0000x reference)
import jax
import jax.numpy as jnp
from jax.experimental import pallas as pl


def kernel(tokens, W_E):
    raise NotImplementedError("write your pallas kernel here")


# speedup vs baseline: 1.2836x; 1.2836x over previous
import functools

import jax
import jax.numpy as jnp
from jax import lax
from jax.experimental import pallas as pl
from jax.experimental.pallas import tpu as pltpu
from jax.experimental.pallas import tpu_sc as plsc

_NC = 2
_NS = 16
_NW = _NC * _NS

_G = 128


def kernel(tokens, W_E):
    B, S = tokens.shape
    V, D = W_E.shape
    N = B * S
    assert N % (_NW * _G) == 0
    ng = N // (_NW * _G)

    idx3 = tokens.reshape(_NW, ng, _G).astype(jnp.int32)
    mesh = plsc.VectorSubcoreMesh(core_axis_name="c", subcore_axis_name="s")

    @functools.partial(
        pl.kernel,
        out_type=jax.ShapeDtypeStruct((N, D), jnp.float32),
        mesh=mesh,
        scratch_types=[
            pltpu.VMEM((ng, _G), jnp.int32),
            pltpu.VMEM((_G, D), jnp.float32),
            pltpu.SemaphoreType.DMA,
        ],
    )
    def emb(idx_hbm, table_hbm, out_hbm, idx_v, rows_v, sem):
        wid = lax.axis_index("s") * _NC + lax.axis_index("c")
        base = wid * (ng * _G)
        pltpu.sync_copy(idx_hbm.at[wid], idx_v)

        def body(g, carry):
            pltpu.async_copy(table_hbm.at[idx_v.at[g]], rows_v, sem).wait()
            pltpu.sync_copy(rows_v, out_hbm.at[pl.ds(base + g * _G, _G)])
            return carry

        lax.fori_loop(0, ng, body, 0)

    out = emb(idx3, W_E)
    return out.reshape(B, S, D)


# speedup vs baseline: 1.7657x; 1.3756x over previous
import functools

import jax
import jax.numpy as jnp
from jax import lax
from jax.experimental import pallas as pl
from jax.experimental.pallas import tpu as pltpu
from jax.experimental.pallas import tpu_sc as plsc

_NC = 2
_NS = 16
_NW = _NC * _NS

_G = 128


def kernel(tokens, W_E):
    B, S = tokens.shape
    V, D = W_E.shape
    N = B * S
    assert N % (_NW * _G * 2) == 0
    ng = N // (_NW * _G)
    npairs = ng // 2

    idx3 = tokens.reshape(_NW, ng, _G).astype(jnp.int32)
    mesh = plsc.VectorSubcoreMesh(core_axis_name="c", subcore_axis_name="s")

    @functools.partial(
        pl.kernel,
        out_type=jax.ShapeDtypeStruct((N, D), jnp.float32),
        mesh=mesh,
        scratch_types=[
            pltpu.VMEM((ng, _G), jnp.int32),
            pltpu.VMEM((2, _G, D), jnp.float32),
            pltpu.SemaphoreType.DMA,
            pltpu.SemaphoreType.DMA,
        ],
    )
    def emb(idx_hbm, table_hbm, out_hbm, idx_v, rows_v, sem0, sem1):
        wid = lax.axis_index("s") * _NC + lax.axis_index("c")
        base = wid * (ng * _G)
        pltpu.sync_copy(idx_hbm.at[wid], idx_v)
        sems = (sem0, sem1)

        def start_gather(g, b):
            pltpu.async_copy(table_hbm.at[idx_v.at[g]], rows_v.at[b], sems[b])

        def wait_gather(g, b):
            pltpu.make_async_copy(
                table_hbm.at[idx_v.at[g]], rows_v.at[b], sems[b]
            ).wait()

        start_gather(0, 0)
        start_gather(1, 1)

        def body(t, carry):
            g = 2 * t
            for b in range(2):
                wait_gather(g + b, b)
                pltpu.sync_copy(
                    rows_v.at[b], out_hbm.at[pl.ds(base + (g + b) * _G, _G)]
                )

                @pl.when(t < npairs - 1)
                def _():
                    start_gather(g + b + 2, b)

            return carry

        lax.fori_loop(0, npairs, body, 0)

    out = emb(idx3, W_E)
    return out.reshape(B, S, D)


# speedup vs baseline: 1.7854x; 1.0111x over previous
import functools

import jax
import jax.numpy as jnp
from jax import lax
from jax.experimental import pallas as pl
from jax.experimental.pallas import tpu as pltpu
from jax.experimental.pallas import tpu_sc as plsc

_NC = 2
_NS = 16
_NW = _NC * _NS

_G = 128


def kernel(tokens, W_E):
    B, S = tokens.shape
    V, D = W_E.shape
    N = B * S
    ng = N // (_NW * _G)
    nb = 5
    assert N % (_NW * _G) == 0 and ng % nb == 0
    nt = ng // nb

    idx3 = tokens.reshape(_NW, ng, _G).astype(jnp.int32)
    mesh = plsc.VectorSubcoreMesh(core_axis_name="c", subcore_axis_name="s")

    @functools.partial(
        pl.kernel,
        out_type=jax.ShapeDtypeStruct((N, D), jnp.float32),
        mesh=mesh,
        scratch_types=[
            pltpu.VMEM((ng, _G), jnp.int32),
            pltpu.VMEM((nb, _G, D), jnp.float32),
            [pltpu.SemaphoreType.DMA] * nb,
        ],
    )
    def emb(idx_hbm, table_hbm, out_hbm, idx_v, rows_v, sems):
        wid = lax.axis_index("s") * _NC + lax.axis_index("c")
        base = wid * (ng * _G)
        pltpu.sync_copy(idx_hbm.at[wid], idx_v)

        def start_gather(g, b):
            pltpu.async_copy(table_hbm.at[idx_v.at[g]], rows_v.at[b], sems[b])

        def wait_gather(g, b):
            pltpu.make_async_copy(
                table_hbm.at[idx_v.at[g]], rows_v.at[b], sems[b]
            ).wait()

        for b in range(nb):
            start_gather(b, b)

        def body(t, carry):
            g = nb * t
            for b in range(nb):
                wait_gather(g + b, b)
                pltpu.sync_copy(
                    rows_v.at[b], out_hbm.at[pl.ds(base + (g + b) * _G, _G)]
                )

                @pl.when(t < nt - 1)
                def _():
                    start_gather(g + b + nb, b)

            return carry

        lax.fori_loop(0, nt, body, 0)

    out = emb(idx3, W_E)
    return out.reshape(B, S, D)
